# baseline (speedup 1.0000x reference)
import functools

import jax
import jax.numpy as jnp
from jax import lax
from jax.experimental import pallas as pl
from jax.experimental.pallas import tpu as pltpu
from jax.experimental.pallas import tpu_sc as plsc

N = 10000
E = 160000
D = 256
H = 256
L = 2
S = 2
G = 64

NP_ = 10240
BN = 1024
NB = NP_ // BN
HH = H // 2
NT = 16
EPT = E // NT
WIN = 128
NW = 79
EPT_P = NW * WIN
RPT = NP_ // NT

_f32 = jnp.float32
_i32 = jnp.int32


_sc_mesh = plsc.VectorSubcoreMesh(core_axis_name="c", subcore_axis_name="s")


@functools.partial(
    pl.kernel,
    out_type=jax.ShapeDtypeStruct((S, NP_), _f32),
    mesh=_sc_mesh,
    scratch_types=[
        pltpu.VMEM_SHARED((NP_,), _f32),
        pltpu.VMEM((NW, WIN), _i32),
        pltpu.VMEM((NW, WIN), _f32),
        pltpu.VMEM((RPT,), _f32),
    ],
)
def _deg_kernel(dst_r, ew_r, deg_out, deg_s, dst_v, ew_v, zb):
    c = lax.axis_index("c")
    sub = lax.axis_index("s")

    def _zb(k, carry):
        zb[pl.ds(k * 16, 16)] = jnp.zeros((16,), _f32)
        return carry

    lax.fori_loop(0, RPT // 16, _zb, 0)
    pltpu.sync_copy(zb, deg_s.at[pl.ds(sub * RPT, RPT)])
    pltpu.sync_copy(dst_r.at[c, sub], dst_v)
    pltpu.sync_copy(ew_r.at[c, sub], ew_v)
    plsc.subcore_barrier()

    def _win(w, carry):
        pltpu.sync_copy(ew_v.at[w], deg_s.at[dst_v.at[w]], add=True)
        return carry

    lax.fori_loop(0, NW, _win, 0)
    plsc.subcore_barrier()
    pltpu.sync_copy(deg_s.at[pl.ds(sub * RPT, RPT)],
                    deg_out.at[c, pl.ds(sub * RPT, RPT)])


@functools.partial(
    pl.kernel,
    out_type=[jax.ShapeDtypeStruct((NP_, HH), _f32) for _ in range(4)],
    mesh=_sc_mesh,
    scratch_types=[
        pltpu.VMEM_SHARED((NP_, HH), _f32),
        pltpu.VMEM((NW, WIN), _i32),
        pltpu.VMEM((2, 2, WIN), _i32),
        pltpu.VMEM((2, WIN, HH), _f32),
        pltpu.SemaphoreType.DMA,
        pltpu.SemaphoreType.DMA,
        pltpu.SemaphoreType.DMA,
        pltpu.SemaphoreType.DMA,
        pltpu.SemaphoreType.DMA,
        pltpu.SemaphoreType.DMA,
    ],
    compiler_params=pltpu.CompilerParams(needs_layout_passes=False),
)
def _msg_kernel(y00, y01, y10, y11, sc_r, dst_r,
                a00, a01, a10, a11,
                acc_s, dst_v, sc_g, msg_v,
                isem0, isem1, gsem0, gsem1, ssem0, ssem1):
    c = lax.axis_index("c")
    sub = lax.axis_index("s")
    isems = (isem0, isem1)
    gsems = (gsem0, gsem1)
    ssems = (ssem0, ssem1)

    for s in range(S):
        def _zb(j, carry):
            for f in range(HH // 16):
                msg_v[0, j, pl.ds(f * 16, 16)] = jnp.zeros((16,), _f32)
            return carry

        lax.fori_loop(0, WIN, _zb, 0)
        for k in range(RPT // WIN):
            pltpu.sync_copy(msg_v.at[0],
                            acc_s.at[pl.ds(sub * RPT + k * WIN, WIN)])
        pltpu.sync_copy(dst_r.at[s, sub], dst_v)
        plsc.subcore_barrier()

        ys = (y00, y01) if s == 0 else (y10, y11)

        def _stage_idx(w, b, sem, s=s):
            pltpu.async_copy(sc_r.at[s, sub, w], sc_g.at[b], sem)

        def _wait_idx(w, b, sem, s=s):
            pltpu.make_async_copy(sc_r.at[s, sub, w], sc_g.at[b],
                                  sem).wait()

        def _gather(b, sem, ys=ys):
            @pl.when(c == 0)
            def _():
                pltpu.async_copy(ys[0].at[sc_g.at[b, 0]], msg_v.at[b], sem)

            @pl.when(c == 1)
            def _():
                pltpu.async_copy(ys[1].at[sc_g.at[b, 0]], msg_v.at[b], sem)

        def _wait_gather(b, sem, ys=ys):
            pltpu.make_async_copy(ys[0].at[sc_g.at[b, 0]], msg_v.at[b],
                                  sem).wait()

        def _scatter(w, b, sem):
            pltpu.async_copy(msg_v.at[b, pl.ds(0, 16)],
                             acc_s.at[dst_v.at[w, pl.ds(0, 16)]], sem,
                             add=True)

        def _wait_scatter(w, b, sem):
            pltpu.make_async_copy(msg_v.at[b, pl.ds(0, 16)],
                                  acc_s.at[dst_v.at[w, pl.ds(0, 16)]],
                                  sem).wait()

        _stage_idx(0, 0, isems[0])
        _wait_idx(0, 0, isems[0])
        _gather(0, gsems[0])
        _stage_idx(1, 1, isems[1])

        def _make_win(b, nb):
            def _win(w, carry):
                w1 = jnp.minimum(w + 1, NW - 1)

                @pl.when(w + 1 < NW)
                def _():
                    _wait_idx(w1, nb, isems[nb])

                    @pl.when(w >= 1)
                    def _():
                        _wait_scatter(w - 1, nb, ssems[nb])

                    _gather(nb, gsems[nb])

                _wait_gather(b, gsems[b])

                @plsc.parallel_loop(0, WIN, step=1, unroll=8)
                def _edge(j):
                    idx = jnp.zeros((16,), _i32) + j
                    cf_i = plsc.load_gather(sc_g.at[b, 1], [idx])
                    cf = plsc.bitcast(cf_i, _f32)
                    for f in range(HH // 16):
                        sl = pl.ds(f * 16, 16)
                        msg_v[b, j, sl] = msg_v[b, j, sl] * cf

                w2 = jnp.minimum(w + 2, NW - 1)

                @pl.when(w + 2 < NW)
                def _():
                    _stage_idx(w2, b, isems[b])

                _scatter(w, b, ssems[b])
                return carry

            return _win

        loop_b = _make_win(0, 1)
        loop_b2 = _make_win(1, 0)

        def _pair(k, carry):
            loop_b(2 * k, 0)
            loop_b2(2 * k + 1, 0)
            return carry

        lax.fori_loop(0, NW // 2, _pair, 0)
        loop_b(NW - 1, 0)
        _wait_scatter(NW - 1, (NW - 1) % 2, ssems[(NW - 1) % 2])
        _wait_scatter(NW - 2, (NW - 2) % 2, ssems[(NW - 2) % 2])
        plsc.subcore_barrier()

        outs = (a00, a01) if s == 0 else (a10, a11)

        @pl.when(c == 0)
        def _():
            pltpu.sync_copy(acc_s.at[pl.ds(sub * RPT, RPT)],
                            outs[0].at[pl.ds(sub * RPT, RPT)])

        @pl.when(c == 1)
        def _():
            pltpu.sync_copy(acc_s.at[pl.ds(sub * RPT, RPT)],
                            outs[1].at[pl.ds(sub * RPT, RPT)])

        plsc.subcore_barrier()


def _k1_body(x_ref, w_ref, deg_ref, y0_ref, y1_ref):
    dinv = lax.rsqrt(deg_ref[...] + 1.0)
    y = dinv * jnp.dot(x_ref[...], w_ref[...],
                       preferred_element_type=_f32)
    y0_ref[...] = y[:, :HH]
    y1_ref[...] = y[:, HH:]


def _tc_layer1(x_pad, w, deg):
    return pl.pallas_call(
        _k1_body,
        grid=(NB,),
        in_specs=[
            pl.BlockSpec((BN, D), lambda i: (i, 0)),
            pl.BlockSpec((D, H), lambda i: (0, 0)),
            pl.BlockSpec((BN, 1), lambda i: (i, 0)),
        ],
        out_specs=[
            pl.BlockSpec((BN, HH), lambda i: (i, 0)),
            pl.BlockSpec((BN, HH), lambda i: (i, 0)),
        ],
        out_shape=[jax.ShapeDtypeStruct((NP_, HH), _f32) for _ in range(2)],
    )(x_pad, w, deg)


def _pool_part(batch_blk, z_blk):
    iota = lax.broadcasted_iota(_i32, (BN, G), 1)
    ind = (batch_blk == iota).astype(_f32)
    return lax.dot_general(ind, z_blk, (((0,), (0,)), ((), ())),
                           preferred_element_type=_f32)


def _k2_body(a0_ref, a1_ref, y0_ref, y1_ref, deg_ref, w2_ref, b1_ref,
             al_ref, batch_ref, y20_ref, y21_ref, g1_ref):
    i = pl.program_id(0)
    dinv = lax.rsqrt(deg_ref[...] + 1.0)
    acc = jnp.concatenate([a0_ref[...], a1_ref[...]], axis=1)
    y1 = jnp.concatenate([y0_ref[...], y1_ref[...]], axis=1)
    z1 = dinv * (acc + y1) + b1_ref[...]
    z1 = jnp.where(z1 >= 0, z1, al_ref[...] * z1)
    y2 = dinv * jnp.dot(z1, w2_ref[...], preferred_element_type=_f32)
    y20_ref[...] = y2[:, :HH]
    y21_ref[...] = y2[:, HH:]
    gp = _pool_part(batch_ref[...], z1)

    @pl.when(i == 0)
    def _():
        g1_ref[...] = gp

    @pl.when(i > 0)
    def _():
        g1_ref[...] += gp


def _tc_layer2(a0, a1, y0, y1, deg, w2, b1, al, batch_col):
    return pl.pallas_call(
        _k2_body,
        grid=(NB,),
        in_specs=[
            pl.BlockSpec((BN, HH), lambda i: (i, 0)),
            pl.BlockSpec((BN, HH), lambda i: (i, 0)),
            pl.BlockSpec((BN, HH), lambda i: (i, 0)),
            pl.BlockSpec((BN, HH), lambda i: (i, 0)),
            pl.BlockSpec((BN, 1), lambda i: (i, 0)),
            pl.BlockSpec((H, H), lambda i: (0, 0)),
            pl.BlockSpec((1, H), lambda i: (0, 0)),
            pl.BlockSpec((1, H), lambda i: (0, 0)),
            pl.BlockSpec((BN, 1), lambda i: (i, 0)),
        ],
        out_specs=[
            pl.BlockSpec((BN, HH), lambda i: (i, 0)),
            pl.BlockSpec((BN, HH), lambda i: (i, 0)),
            pl.BlockSpec((G, H), lambda i: (0, 0)),
        ],
        out_shape=[
            jax.ShapeDtypeStruct((NP_, HH), _f32),
            jax.ShapeDtypeStruct((NP_, HH), _f32),
            jax.ShapeDtypeStruct((G, H), _f32),
        ],
    )(a0, a1, y0, y1, deg, w2, b1, al, batch_col)


def _k3_body(a0_ref, a1_ref, y0_ref, y1_ref, deg_ref, b2_ref, al_ref,
             batch_ref, z2_ref, g2_ref):
    i = pl.program_id(0)
    dinv = lax.rsqrt(deg_ref[...] + 1.0)
    acc = jnp.concatenate([a0_ref[...], a1_ref[...]], axis=1)
    y2 = jnp.concatenate([y0_ref[...], y1_ref[...]], axis=1)
    z2 = dinv * (acc + y2) + b2_ref[...]
    z2 = jnp.where(z2 >= 0, z2, al_ref[...] * z2)
    z2_ref[...] = z2
    gp = _pool_part(batch_ref[...], z2)

    @pl.when(i == 0)
    def _():
        g2_ref[...] = gp

    @pl.when(i > 0)
    def _():
        g2_ref[...] += gp


def _tc_layer3(a0, a1, y0, y1, deg, b2, al, batch_col):
    return pl.pallas_call(
        _k3_body,
        grid=(NB,),
        in_specs=[
            pl.BlockSpec((BN, HH), lambda i: (i, 0)),
            pl.BlockSpec((BN, HH), lambda i: (i, 0)),
            pl.BlockSpec((BN, HH), lambda i: (i, 0)),
            pl.BlockSpec((BN, HH), lambda i: (i, 0)),
            pl.BlockSpec((BN, 1), lambda i: (i, 0)),
            pl.BlockSpec((1, H), lambda i: (0, 0)),
            pl.BlockSpec((1, H), lambda i: (0, 0)),
            pl.BlockSpec((BN, 1), lambda i: (i, 0)),
        ],
        out_specs=[
            pl.BlockSpec((BN, H), lambda i: (i, 0)),
            pl.BlockSpec((G, H), lambda i: (0, 0)),
        ],
        out_shape=[
            jax.ShapeDtypeStruct((NP_, H), _f32),
            jax.ShapeDtypeStruct((G, H), _f32),
        ],
    )(a0, a1, y0, y1, deg, b2, al, batch_col)


def kernel(batch, x, edge_index, edge_weight, params):
    pad = ((0, 0), (0, 0), (0, EPT_P - EPT))
    src_p = jnp.pad(edge_index[:, 0, :].reshape(S, NT, EPT), pad)
    dst_p = jnp.pad(edge_index[:, 1, :].reshape(S, NT, EPT), pad)
    ew_p = jnp.pad(edge_weight.reshape(S, NT, EPT), pad)
    src_r = src_p.reshape(S, NT, NW, WIN).astype(_i32)
    dst_r = dst_p.reshape(S, NT, NW, WIN).astype(_i32)
    ew_r = ew_p.reshape(S, NT, NW, WIN).astype(_f32)
    sc_r = jnp.stack(
        [src_r, lax.bitcast_convert_type(ew_r, _i32)], axis=3)

    x_pad = jnp.zeros((NP_, D), _f32).at[:N].set(x)
    batch_col = jnp.full((NP_, 1), G, _i32).at[:N, 0].set(batch.astype(_i32))

    deg = _deg_kernel(dst_r, ew_r)
    deg_col = [deg[t].reshape(NP_, 1) for t in range(S)]

    y1 = [
        _tc_layer1(x_pad, params[t]["W"][0], deg_col[t]) for t in range(S)
    ]
    a1 = _msg_kernel(y1[0][0], y1[0][1], y1[1][0], y1[1][1],
                     sc_r, dst_r)

    b_row = [[params[t]["b"][l].reshape(1, H) for l in range(L)]
             for t in range(S)]
    al_row = [params[t]["alpha"].reshape(1, H) for t in range(S)]
    k2 = [
        _tc_layer2(a1[2 * t], a1[2 * t + 1], y1[t][0], y1[t][1],
                   deg_col[t], params[t]["W"][1], b_row[t][0], al_row[t],
                   batch_col)
        for t in range(S)
    ]
    a2 = _msg_kernel(k2[0][0], k2[0][1], k2[1][0], k2[1][1],
                     sc_r, dst_r)

    k3 = [
        _tc_layer3(a2[2 * t], a2[2 * t + 1], k2[t][0], k2[t][1],
                   deg_col[t], b_row[t][1], al_row[t], batch_col)
        for t in range(S)
    ]

    z_T = jnp.stack([k3[t][0][:N] for t in range(S)])
    g_T = jnp.stack([jnp.concatenate([k2[t][2], k3[t][1]], axis=1)
                     for t in range(S)])
    return (z_T, g_T)


# speedup vs baseline: 1.0263x; 1.0263x over previous
import functools

import jax
import jax.numpy as jnp
from jax import lax
from jax.experimental import pallas as pl
from jax.experimental.pallas import tpu as pltpu
from jax.experimental.pallas import tpu_sc as plsc

N = 10000
E = 160000
D = 256
H = 256
L = 2
S = 2
G = 64

NP_ = 10240
BN = 1024
NB = NP_ // BN
HH = H // 2
NT = 16
EPT = E // NT
WIN = 128
NW = 79
EPT_P = NW * WIN
RPT = NP_ // NT

_f32 = jnp.float32
_i32 = jnp.int32


_sc_mesh = plsc.VectorSubcoreMesh(core_axis_name="c", subcore_axis_name="s")


@functools.partial(
    pl.kernel,
    out_type=jax.ShapeDtypeStruct((S, NP_), _f32),
    mesh=_sc_mesh,
    scratch_types=[
        pltpu.VMEM_SHARED((NP_,), _f32),
        pltpu.VMEM((NW, WIN), _i32),
        pltpu.VMEM((NW, WIN), _f32),
        pltpu.VMEM((RPT,), _f32),
    ],
)
def _deg_kernel(dst_r, ew_r, deg_out, deg_s, dst_v, ew_v, zb):
    c = lax.axis_index("c")
    sub = lax.axis_index("s")

    def _zb(k, carry):
        zb[pl.ds(k * 16, 16)] = jnp.zeros((16,), _f32)
        return carry

    lax.fori_loop(0, RPT // 16, _zb, 0)
    pltpu.sync_copy(zb, deg_s.at[pl.ds(sub * RPT, RPT)])
    pltpu.sync_copy(dst_r.at[c, sub], dst_v)
    pltpu.sync_copy(ew_r.at[c, sub], ew_v)
    plsc.subcore_barrier()

    def _win(w, carry):
        pltpu.sync_copy(ew_v.at[w], deg_s.at[dst_v.at[w]], add=True)
        return carry

    lax.fori_loop(0, NW, _win, 0)
    plsc.subcore_barrier()
    pltpu.sync_copy(deg_s.at[pl.ds(sub * RPT, RPT)],
                    deg_out.at[c, pl.ds(sub * RPT, RPT)])


@functools.partial(
    pl.kernel,
    out_type=[jax.ShapeDtypeStruct((NP_, HH), _f32) for _ in range(4)],
    mesh=_sc_mesh,
    scratch_types=[
        pltpu.VMEM_SHARED((NP_, HH), _f32),
        pltpu.VMEM((NW, WIN), _i32),
        pltpu.VMEM((2, 2, WIN), _i32),
        pltpu.VMEM((2, WIN, HH // 2), _i32),
        pltpu.VMEM((WIN, HH), _f32),
        pltpu.SemaphoreType.DMA,
        pltpu.SemaphoreType.DMA,
        pltpu.SemaphoreType.DMA,
        pltpu.SemaphoreType.DMA,
        pltpu.SemaphoreType.DMA,
    ],
    compiler_params=pltpu.CompilerParams(needs_layout_passes=False,
                                         use_tc_tiling_on_sc=False),
)
def _msg_kernel(y00, y01, y10, y11, sc_r, dst_r,
                a00, a01, a10, a11,
                acc_s, dst_v, sc_g, gbuf, msg_f,
                isem0, isem1, gsem0, gsem1, ssem):
    c = lax.axis_index("c")
    sub = lax.axis_index("s")
    isems = (isem0, isem1)
    gsems = (gsem0, gsem1)
    himask = jnp.full((16,), -65536, _i32)

    for s in range(S):
        def _zb(j, carry):
            for f in range(HH // 16):
                msg_f[j, pl.ds(f * 16, 16)] = jnp.zeros((16,), _f32)
            return carry

        lax.fori_loop(0, WIN, _zb, 0)
        for k in range(RPT // WIN):
            pltpu.sync_copy(msg_f,
                            acc_s.at[pl.ds(sub * RPT + k * WIN, WIN)])
        pltpu.sync_copy(dst_r.at[s, sub], dst_v)
        plsc.subcore_barrier()

        ys = (y00, y01) if s == 0 else (y10, y11)

        def _stage_idx(w, b, sem, s=s):
            pltpu.async_copy(sc_r.at[s, sub, w], sc_g.at[b], sem)

        def _wait_idx(w, b, sem, s=s):
            pltpu.make_async_copy(sc_r.at[s, sub, w], sc_g.at[b],
                                  sem).wait()

        def _gather(b, sem, ys=ys):
            @pl.when(c == 0)
            def _():
                pltpu.async_copy(ys[0].at[sc_g.at[b, 0]], gbuf.at[b], sem)

            @pl.when(c == 1)
            def _():
                pltpu.async_copy(ys[1].at[sc_g.at[b, 0]], gbuf.at[b], sem)

        def _wait_gather(b, sem, ys=ys):
            pltpu.make_async_copy(ys[0].at[sc_g.at[b, 0]], gbuf.at[b],
                                  sem).wait()

        def _scatter(w, sem):
            pltpu.async_copy(msg_f, acc_s.at[dst_v.at[w]], sem, add=True)

        def _wait_scatter(w, sem):
            pltpu.make_async_copy(msg_f, acc_s.at[dst_v.at[w]],
                                  sem).wait()

        _stage_idx(0, 0, isems[0])
        _wait_idx(0, 0, isems[0])
        _gather(0, gsems[0])
        _stage_idx(1, 1, isems[1])

        def _make_win(b, nb):
            def _win(w, carry):
                w1 = jnp.minimum(w + 1, NW - 1)

                @pl.when(w + 1 < NW)
                def _():
                    _wait_idx(w1, nb, isems[nb])
                    _gather(nb, gsems[nb])

                _wait_gather(b, gsems[b])

                @pl.when(w >= 1)
                def _():
                    _wait_scatter(w - 1, ssem)

                @plsc.parallel_loop(0, WIN, step=1, unroll=8)
                def _edge(j):
                    idx = jnp.zeros((16,), _i32) + j
                    cf_i = plsc.load_gather(sc_g.at[b, 1], [idx])
                    cf = plsc.bitcast(cf_i, _f32)
                    for ch in range(HH // 32):
                        wrd = gbuf[b, j, pl.ds(ch * 16, 16)]
                        lo = plsc.bitcast(
                            lax.shift_left(wrd, jnp.full((16,), 16, _i32)),
                            _f32)
                        hi = plsc.bitcast(wrd & himask, _f32)
                        msg_f[j, pl.ds(ch * 16, 16)] = lo * cf
                        msg_f[j, pl.ds(64 + ch * 16, 16)] = hi * cf

                w2 = jnp.minimum(w + 2, NW - 1)

                @pl.when(w + 2 < NW)
                def _():
                    _stage_idx(w2, b, isems[b])

                _scatter(w, ssem)
                return carry

            return _win

        loop_b = _make_win(0, 1)
        loop_b2 = _make_win(1, 0)

        def _pair(k, carry):
            loop_b(2 * k, 0)
            loop_b2(2 * k + 1, 0)
            return carry

        lax.fori_loop(0, NW // 2, _pair, 0)
        loop_b(NW - 1, 0)
        _wait_scatter(NW - 1, ssem)
        plsc.subcore_barrier()

        outs = (a00, a01) if s == 0 else (a10, a11)

        @pl.when(c == 0)
        def _():
            pltpu.sync_copy(acc_s.at[pl.ds(sub * RPT, RPT)],
                            outs[0].at[pl.ds(sub * RPT, RPT)])

        @pl.when(c == 1)
        def _():
            pltpu.sync_copy(acc_s.at[pl.ds(sub * RPT, RPT)],
                            outs[1].at[pl.ds(sub * RPT, RPT)])

        plsc.subcore_barrier()


def _rne_bf16_bits(x):
    u = lax.bitcast_convert_type(x, _i32)
    lsb = lax.shift_right_logical(u, 16) & 1
    return lax.shift_right_logical(u + 32767 + lsb, 16)


def _inter_bf16(yh):
    ra = _rne_bf16_bits(yh[:, :HH // 2])
    rb = _rne_bf16_bits(yh[:, HH // 2:])
    return ra | lax.shift_left(rb, 16)


def _k1_body(x_ref, w_ref, deg_ref, y_ref, yb0_ref, yb1_ref):
    dinv = lax.rsqrt(deg_ref[...] + 1.0)
    y = dinv * jnp.dot(x_ref[...], w_ref[...],
                       preferred_element_type=_f32)
    y_ref[...] = y
    yb0_ref[...] = _inter_bf16(y[:, :HH])
    yb1_ref[...] = _inter_bf16(y[:, HH:])


def _tc_layer1(x_pad, w, deg):
    return pl.pallas_call(
        _k1_body,
        grid=(NB,),
        in_specs=[
            pl.BlockSpec((BN, D), lambda i: (i, 0)),
            pl.BlockSpec((D, H), lambda i: (0, 0)),
            pl.BlockSpec((BN, 1), lambda i: (i, 0)),
        ],
        out_specs=[
            pl.BlockSpec((BN, H), lambda i: (i, 0)),
            pl.BlockSpec((BN, HH // 2), lambda i: (i, 0)),
            pl.BlockSpec((BN, HH // 2), lambda i: (i, 0)),
        ],
        out_shape=[
            jax.ShapeDtypeStruct((NP_, H), _f32),
            jax.ShapeDtypeStruct((NP_, HH // 2), _i32),
            jax.ShapeDtypeStruct((NP_, HH // 2), _i32),
        ],
    )(x_pad, w, deg)


def _pool_part(batch_blk, z_blk):
    iota = lax.broadcasted_iota(_i32, (BN, G), 1)
    ind = (batch_blk == iota).astype(_f32)
    return lax.dot_general(ind, z_blk, (((0,), (0,)), ((), ())),
                           preferred_element_type=_f32)


def _k2_body(a0_ref, a1_ref, y_ref, deg_ref, w2_ref, b1_ref,
             al_ref, batch_ref, y2_ref, y2b0_ref, y2b1_ref, g1_ref):
    i = pl.program_id(0)
    dinv = lax.rsqrt(deg_ref[...] + 1.0)
    acc = jnp.concatenate([a0_ref[...], a1_ref[...]], axis=1)
    z1 = dinv * (acc + y_ref[...]) + b1_ref[...]
    z1 = jnp.where(z1 >= 0, z1, al_ref[...] * z1)
    y2 = dinv * jnp.dot(z1, w2_ref[...], preferred_element_type=_f32)
    y2_ref[...] = y2
    y2b0_ref[...] = _inter_bf16(y2[:, :HH])
    y2b1_ref[...] = _inter_bf16(y2[:, HH:])
    gp = _pool_part(batch_ref[...], z1)

    @pl.when(i == 0)
    def _():
        g1_ref[...] = gp

    @pl.when(i > 0)
    def _():
        g1_ref[...] += gp


def _tc_layer2(a0, a1, y, deg, w2, b1, al, batch_col):
    return pl.pallas_call(
        _k2_body,
        grid=(NB,),
        in_specs=[
            pl.BlockSpec((BN, HH), lambda i: (i, 0)),
            pl.BlockSpec((BN, HH), lambda i: (i, 0)),
            pl.BlockSpec((BN, H), lambda i: (i, 0)),
            pl.BlockSpec((BN, 1), lambda i: (i, 0)),
            pl.BlockSpec((H, H), lambda i: (0, 0)),
            pl.BlockSpec((1, H), lambda i: (0, 0)),
            pl.BlockSpec((1, H), lambda i: (0, 0)),
            pl.BlockSpec((BN, 1), lambda i: (i, 0)),
        ],
        out_specs=[
            pl.BlockSpec((BN, H), lambda i: (i, 0)),
            pl.BlockSpec((BN, HH // 2), lambda i: (i, 0)),
            pl.BlockSpec((BN, HH // 2), lambda i: (i, 0)),
            pl.BlockSpec((G, H), lambda i: (0, 0)),
        ],
        out_shape=[
            jax.ShapeDtypeStruct((NP_, H), _f32),
            jax.ShapeDtypeStruct((NP_, HH // 2), _i32),
            jax.ShapeDtypeStruct((NP_, HH // 2), _i32),
            jax.ShapeDtypeStruct((G, H), _f32),
        ],
    )(a0, a1, y, deg, w2, b1, al, batch_col)


def _k3_body(a0_ref, a1_ref, y_ref, deg_ref, b2_ref, al_ref,
             batch_ref, z2_ref, g2_ref):
    i = pl.program_id(0)
    dinv = lax.rsqrt(deg_ref[...] + 1.0)
    acc = jnp.concatenate([a0_ref[...], a1_ref[...]], axis=1)
    z2 = dinv * (acc + y_ref[...]) + b2_ref[...]
    z2 = jnp.where(z2 >= 0, z2, al_ref[...] * z2)
    z2_ref[...] = z2
    gp = _pool_part(batch_ref[...], z2)

    @pl.when(i == 0)
    def _():
        g2_ref[...] = gp

    @pl.when(i > 0)
    def _():
        g2_ref[...] += gp


def _tc_layer3(a0, a1, y, deg, b2, al, batch_col):
    return pl.pallas_call(
        _k3_body,
        grid=(NB,),
        in_specs=[
            pl.BlockSpec((BN, HH), lambda i: (i, 0)),
            pl.BlockSpec((BN, HH), lambda i: (i, 0)),
            pl.BlockSpec((BN, H), lambda i: (i, 0)),
            pl.BlockSpec((BN, 1), lambda i: (i, 0)),
            pl.BlockSpec((1, H), lambda i: (0, 0)),
            pl.BlockSpec((1, H), lambda i: (0, 0)),
            pl.BlockSpec((BN, 1), lambda i: (i, 0)),
        ],
        out_specs=[
            pl.BlockSpec((BN, H), lambda i: (i, 0)),
            pl.BlockSpec((G, H), lambda i: (0, 0)),
        ],
        out_shape=[
            jax.ShapeDtypeStruct((NP_, H), _f32),
            jax.ShapeDtypeStruct((G, H), _f32),
        ],
    )(a0, a1, y, deg, b2, al, batch_col)


def kernel(batch, x, edge_index, edge_weight, params):
    pad = ((0, 0), (0, 0), (0, EPT_P - EPT))
    src_p = jnp.pad(edge_index[:, 0, :].reshape(S, NT, EPT), pad)
    dst_p = jnp.pad(edge_index[:, 1, :].reshape(S, NT, EPT), pad)
    ew_p = jnp.pad(edge_weight.reshape(S, NT, EPT), pad)
    src_r = src_p.reshape(S, NT, NW, WIN).astype(_i32)
    dst_r = dst_p.reshape(S, NT, NW, WIN).astype(_i32)
    ew_r = ew_p.reshape(S, NT, NW, WIN).astype(_f32)
    sc_r = jnp.stack(
        [src_r, lax.bitcast_convert_type(ew_r, _i32)], axis=3)

    x_pad = jnp.zeros((NP_, D), _f32).at[:N].set(x)
    batch_col = jnp.full((NP_, 1), G, _i32).at[:N, 0].set(batch.astype(_i32))

    deg = _deg_kernel(dst_r, ew_r)
    deg_col = [deg[t].reshape(NP_, 1) for t in range(S)]

    y1 = [
        _tc_layer1(x_pad, params[t]["W"][0], deg_col[t]) for t in range(S)
    ]
    a1 = _msg_kernel(y1[0][1], y1[0][2], y1[1][1], y1[1][2],
                     sc_r, dst_r)

    b_row = [[params[t]["b"][l].reshape(1, H) for l in range(L)]
             for t in range(S)]
    al_row = [params[t]["alpha"].reshape(1, H) for t in range(S)]
    k2 = [
        _tc_layer2(a1[2 * t], a1[2 * t + 1], y1[t][0],
                   deg_col[t], params[t]["W"][1], b_row[t][0], al_row[t],
                   batch_col)
        for t in range(S)
    ]
    a2 = _msg_kernel(k2[0][1], k2[0][2], k2[1][1], k2[1][2],
                     sc_r, dst_r)

    k3 = [
        _tc_layer3(a2[2 * t], a2[2 * t + 1], k2[t][0],
                   deg_col[t], b_row[t][1], al_row[t], batch_col)
        for t in range(S)
    ]

    z_T = jnp.stack([k3[t][0][:N] for t in range(S)])
    g_T = jnp.stack([jnp.concatenate([k2[t][3], k3[t][1]], axis=1)
                     for t in range(S)])
    return (z_T, g_T)


# speedup vs baseline: 1.1285x; 1.0996x over previous
import functools

import jax
import jax.numpy as jnp
from jax import lax
from jax.experimental import pallas as pl
from jax.experimental.pallas import tpu as pltpu
from jax.experimental.pallas import tpu_sc as plsc

N = 10000
E = 160000
D = 256
H = 256
L = 2
S = 2
G = 64

NP_ = 10240
BN = 1024
NB = NP_ // BN
HH = H // 2
NT = 16
EPT = E // NT
WIN = 128
NW = 79
EPT_P = NW * WIN
RPT = NP_ // NT

_f32 = jnp.float32
_i32 = jnp.int32


_sc_mesh = plsc.VectorSubcoreMesh(core_axis_name="c", subcore_axis_name="s")


@functools.partial(
    pl.kernel,
    out_type=jax.ShapeDtypeStruct((S, NP_), _f32),
    mesh=_sc_mesh,
    scratch_types=[
        pltpu.VMEM_SHARED((NP_,), _f32),
        pltpu.VMEM((NW, WIN), _i32),
        pltpu.VMEM((NW, WIN), _f32),
        pltpu.VMEM((RPT,), _f32),
    ],
)
def _deg_kernel(dst_r, ew_r, deg_out, deg_s, dst_v, ew_v, zb):
    c = lax.axis_index("c")
    sub = lax.axis_index("s")

    def _zb(k, carry):
        zb[pl.ds(k * 16, 16)] = jnp.zeros((16,), _f32)
        return carry

    lax.fori_loop(0, RPT // 16, _zb, 0)
    pltpu.sync_copy(zb, deg_s.at[pl.ds(sub * RPT, RPT)])
    pltpu.sync_copy(dst_r.at[c, sub], dst_v)
    pltpu.sync_copy(ew_r.at[c, sub], ew_v)
    plsc.subcore_barrier()

    def _win(w, carry):
        pltpu.sync_copy(ew_v.at[w], deg_s.at[dst_v.at[w]], add=True)
        return carry

    lax.fori_loop(0, NW, _win, 0)
    plsc.subcore_barrier()
    pltpu.sync_copy(deg_s.at[pl.ds(sub * RPT, RPT)],
                    deg_out.at[c, pl.ds(sub * RPT, RPT)])


def _make_msg_kernel(s):
    @functools.partial(
        pl.kernel,
        out_type=[jax.ShapeDtypeStruct((NP_, HH), _f32) for _ in range(2)],
        mesh=_sc_mesh,
        scratch_types=[
            pltpu.VMEM_SHARED((NP_, HH), _f32),
            pltpu.VMEM((NW, WIN), _i32),
            pltpu.VMEM((2, 2, WIN), _i32),
            pltpu.VMEM((2, WIN, HH // 2), _i32),
            pltpu.VMEM((WIN, HH), _f32),
            pltpu.SemaphoreType.DMA,
            pltpu.SemaphoreType.DMA,
            pltpu.SemaphoreType.DMA,
            pltpu.SemaphoreType.DMA,
            pltpu.SemaphoreType.DMA,
        ],
        compiler_params=pltpu.CompilerParams(needs_layout_passes=False,
                                             use_tc_tiling_on_sc=False),
        name=f"msg_scale{s}",
    )
    def _msg_kernel(y0, y1, sc_r, dst_r, a0, a1,
                    acc_s, dst_v, sc_g, gbuf, msg_f,
                    isem0, isem1, gsem0, gsem1, ssem):
        c = lax.axis_index("c")
        sub = lax.axis_index("s")
        isems = (isem0, isem1)
        gsems = (gsem0, gsem1)
        himask = jnp.full((16,), -65536, _i32)

        def _zb(j, carry):
            for f in range(HH // 16):
                msg_f[j, pl.ds(f * 16, 16)] = jnp.zeros((16,), _f32)
            return carry

        lax.fori_loop(0, WIN, _zb, 0)
        for k in range(RPT // WIN):
            pltpu.sync_copy(msg_f,
                            acc_s.at[pl.ds(sub * RPT + k * WIN, WIN)])
        pltpu.sync_copy(dst_r.at[s, sub], dst_v)
        plsc.subcore_barrier()

        def _stage_idx(w, b, sem):
            pltpu.async_copy(sc_r.at[s, sub, w], sc_g.at[b], sem)

        def _wait_idx(w, b, sem):
            pltpu.make_async_copy(sc_r.at[s, sub, w], sc_g.at[b],
                                  sem).wait()

        def _gather(b, sem):
            @pl.when(c == 0)
            def _():
                pltpu.async_copy(y0.at[sc_g.at[b, 0]], gbuf.at[b], sem)

            @pl.when(c == 1)
            def _():
                pltpu.async_copy(y1.at[sc_g.at[b, 0]], gbuf.at[b], sem)

        def _wait_gather(b, sem):
            pltpu.make_async_copy(y0.at[sc_g.at[b, 0]], gbuf.at[b],
                                  sem).wait()

        def _scatter(w, sem):
            pltpu.async_copy(msg_f, acc_s.at[dst_v.at[w]], sem, add=True)

        def _wait_scatter(w, sem):
            pltpu.make_async_copy(msg_f, acc_s.at[dst_v.at[w]],
                                  sem).wait()

        _stage_idx(0, 0, isems[0])
        _wait_idx(0, 0, isems[0])
        _gather(0, gsems[0])
        _stage_idx(1, 1, isems[1])

        def _make_win(b, nb):
            def _win(w, carry):
                w1 = jnp.minimum(w + 1, NW - 1)

                @pl.when(w + 1 < NW)
                def _():
                    _wait_idx(w1, nb, isems[nb])
                    _gather(nb, gsems[nb])

                _wait_gather(b, gsems[b])

                @pl.when(w >= 1)
                def _():
                    _wait_scatter(w - 1, ssem)

                @plsc.parallel_loop(0, WIN, step=1, unroll=8)
                def _edge(j):
                    idx = jnp.zeros((16,), _i32) + j
                    cf_i = plsc.load_gather(sc_g.at[b, 1], [idx])
                    cf = plsc.bitcast(cf_i, _f32)
                    for ch in range(HH // 32):
                        wrd = gbuf[b, j, pl.ds(ch * 16, 16)]
                        lo = plsc.bitcast(
                            lax.shift_left(wrd, jnp.full((16,), 16, _i32)),
                            _f32)
                        hi = plsc.bitcast(wrd & himask, _f32)
                        msg_f[j, pl.ds(ch * 16, 16)] = lo * cf
                        msg_f[j, pl.ds(64 + ch * 16, 16)] = hi * cf

                w2 = jnp.minimum(w + 2, NW - 1)

                @pl.when(w + 2 < NW)
                def _():
                    _stage_idx(w2, b, isems[b])

                _scatter(w, ssem)
                return carry

            return _win

        loop_b = _make_win(0, 1)
        loop_b2 = _make_win(1, 0)

        def _pair(k, carry):
            loop_b(2 * k, 0)
            loop_b2(2 * k + 1, 0)
            return carry

        lax.fori_loop(0, NW // 2, _pair, 0)
        loop_b(NW - 1, 0)
        _wait_scatter(NW - 1, ssem)
        plsc.subcore_barrier()

        @pl.when(c == 0)
        def _():
            pltpu.sync_copy(acc_s.at[pl.ds(sub * RPT, RPT)],
                            a0.at[pl.ds(sub * RPT, RPT)])

        @pl.when(c == 1)
        def _():
            pltpu.sync_copy(acc_s.at[pl.ds(sub * RPT, RPT)],
                            a1.at[pl.ds(sub * RPT, RPT)])

    return _msg_kernel


_msg_kernels = [_make_msg_kernel(0), _make_msg_kernel(1)]


def _rne_bf16_bits(x):
    u = lax.bitcast_convert_type(x, _i32)
    lsb = lax.shift_right_logical(u, 16) & 1
    return lax.shift_right_logical(u + 32767 + lsb, 16)


def _inter_bf16(yh):
    ra = _rne_bf16_bits(yh[:, :HH // 2])
    rb = _rne_bf16_bits(yh[:, HH // 2:])
    return ra | lax.shift_left(rb, 16)


def _k1_body(x_ref, w_ref, deg_ref, y_ref, yb0_ref, yb1_ref):
    dinv = lax.rsqrt(deg_ref[...] + 1.0)
    y = dinv * jnp.dot(x_ref[...], w_ref[...],
                       preferred_element_type=_f32)
    y_ref[...] = y
    yb0_ref[...] = _inter_bf16(y[:, :HH])
    yb1_ref[...] = _inter_bf16(y[:, HH:])


def _tc_layer1(x_pad, w, deg):
    return pl.pallas_call(
        _k1_body,
        grid=(NB,),
        in_specs=[
            pl.BlockSpec((BN, D), lambda i: (i, 0)),
            pl.BlockSpec((D, H), lambda i: (0, 0)),
            pl.BlockSpec((BN, 1), lambda i: (i, 0)),
        ],
        out_specs=[
            pl.BlockSpec((BN, H), lambda i: (i, 0)),
            pl.BlockSpec((BN, HH // 2), lambda i: (i, 0)),
            pl.BlockSpec((BN, HH // 2), lambda i: (i, 0)),
        ],
        out_shape=[
            jax.ShapeDtypeStruct((NP_, H), _f32),
            jax.ShapeDtypeStruct((NP_, HH // 2), _i32),
            jax.ShapeDtypeStruct((NP_, HH // 2), _i32),
        ],
    )(x_pad, w, deg)


def _pool_part(batch_blk, z_blk):
    iota = lax.broadcasted_iota(_i32, (BN, G), 1)
    ind = (batch_blk == iota).astype(_f32)
    return lax.dot_general(ind, z_blk, (((0,), (0,)), ((), ())),
                           preferred_element_type=_f32)


def _k2_body(a0_ref, a1_ref, y_ref, deg_ref, w2_ref, b1_ref,
             al_ref, batch_ref, y2_ref, y2b0_ref, y2b1_ref, g1_ref):
    i = pl.program_id(0)
    dinv = lax.rsqrt(deg_ref[...] + 1.0)
    acc = jnp.concatenate([a0_ref[...], a1_ref[...]], axis=1)
    z1 = dinv * (acc + y_ref[...]) + b1_ref[...]
    z1 = jnp.where(z1 >= 0, z1, al_ref[...] * z1)
    y2 = dinv * jnp.dot(z1, w2_ref[...], preferred_element_type=_f32)
    y2_ref[...] = y2
    y2b0_ref[...] = _inter_bf16(y2[:, :HH])
    y2b1_ref[...] = _inter_bf16(y2[:, HH:])
    gp = _pool_part(batch_ref[...], z1)

    @pl.when(i == 0)
    def _():
        g1_ref[...] = gp

    @pl.when(i > 0)
    def _():
        g1_ref[...] += gp


def _tc_layer2(a0, a1, y, deg, w2, b1, al, batch_col):
    return pl.pallas_call(
        _k2_body,
        grid=(NB,),
        in_specs=[
            pl.BlockSpec((BN, HH), lambda i: (i, 0)),
            pl.BlockSpec((BN, HH), lambda i: (i, 0)),
            pl.BlockSpec((BN, H), lambda i: (i, 0)),
            pl.BlockSpec((BN, 1), lambda i: (i, 0)),
            pl.BlockSpec((H, H), lambda i: (0, 0)),
            pl.BlockSpec((1, H), lambda i: (0, 0)),
            pl.BlockSpec((1, H), lambda i: (0, 0)),
            pl.BlockSpec((BN, 1), lambda i: (i, 0)),
        ],
        out_specs=[
            pl.BlockSpec((BN, H), lambda i: (i, 0)),
            pl.BlockSpec((BN, HH // 2), lambda i: (i, 0)),
            pl.BlockSpec((BN, HH // 2), lambda i: (i, 0)),
            pl.BlockSpec((G, H), lambda i: (0, 0)),
        ],
        out_shape=[
            jax.ShapeDtypeStruct((NP_, H), _f32),
            jax.ShapeDtypeStruct((NP_, HH // 2), _i32),
            jax.ShapeDtypeStruct((NP_, HH // 2), _i32),
            jax.ShapeDtypeStruct((G, H), _f32),
        ],
    )(a0, a1, y, deg, w2, b1, al, batch_col)


def _k3_body(a0_ref, a1_ref, y_ref, deg_ref, b2_ref, al_ref,
             batch_ref, z2_ref, g2_ref):
    i = pl.program_id(0)
    dinv = lax.rsqrt(deg_ref[...] + 1.0)
    acc = jnp.concatenate([a0_ref[...], a1_ref[...]], axis=1)
    z2 = dinv * (acc + y_ref[...]) + b2_ref[...]
    z2 = jnp.where(z2 >= 0, z2, al_ref[...] * z2)
    z2_ref[...] = z2
    gp = _pool_part(batch_ref[...], z2)

    @pl.when(i == 0)
    def _():
        g2_ref[...] = gp

    @pl.when(i > 0)
    def _():
        g2_ref[...] += gp


def _tc_layer3(a0, a1, y, deg, b2, al, batch_col):
    return pl.pallas_call(
        _k3_body,
        grid=(NB,),
        in_specs=[
            pl.BlockSpec((BN, HH), lambda i: (i, 0)),
            pl.BlockSpec((BN, HH), lambda i: (i, 0)),
            pl.BlockSpec((BN, H), lambda i: (i, 0)),
            pl.BlockSpec((BN, 1), lambda i: (i, 0)),
            pl.BlockSpec((1, H), lambda i: (0, 0)),
            pl.BlockSpec((1, H), lambda i: (0, 0)),
            pl.BlockSpec((BN, 1), lambda i: (i, 0)),
        ],
        out_specs=[
            pl.BlockSpec((BN, H), lambda i: (i, 0)),
            pl.BlockSpec((G, H), lambda i: (0, 0)),
        ],
        out_shape=[
            jax.ShapeDtypeStruct((NP_, H), _f32),
            jax.ShapeDtypeStruct((G, H), _f32),
        ],
    )(a0, a1, y, deg, b2, al, batch_col)


def kernel(batch, x, edge_index, edge_weight, params):
    pad = ((0, 0), (0, 0), (0, EPT_P - EPT))
    src_p = jnp.pad(edge_index[:, 0, :].reshape(S, NT, EPT), pad)
    dst_p = jnp.pad(edge_index[:, 1, :].reshape(S, NT, EPT), pad)
    ew_p = jnp.pad(edge_weight.reshape(S, NT, EPT), pad)
    src_r = src_p.reshape(S, NT, NW, WIN).astype(_i32)
    dst_r = dst_p.reshape(S, NT, NW, WIN).astype(_i32)
    ew_r = ew_p.reshape(S, NT, NW, WIN).astype(_f32)
    sc_r = jnp.stack(
        [src_r, lax.bitcast_convert_type(ew_r, _i32)], axis=3)

    x_pad = jnp.zeros((NP_, D), _f32).at[:N].set(x)
    batch_col = jnp.full((NP_, 1), G, _i32).at[:N, 0].set(batch.astype(_i32))

    deg = _deg_kernel(dst_r, ew_r)
    deg_col = [deg[t].reshape(NP_, 1) for t in range(S)]

    y1 = [
        _tc_layer1(x_pad, params[t]["W"][0], deg_col[t]) for t in range(S)
    ]
    a1 = [
        _msg_kernels[t](y1[t][1], y1[t][2], sc_r, dst_r) for t in range(S)
    ]

    b_row = [[params[t]["b"][l].reshape(1, H) for l in range(L)]
             for t in range(S)]
    al_row = [params[t]["alpha"].reshape(1, H) for t in range(S)]
    k2 = [
        _tc_layer2(a1[t][0], a1[t][1], y1[t][0],
                   deg_col[t], params[t]["W"][1], b_row[t][0], al_row[t],
                   batch_col)
        for t in range(S)
    ]
    a2 = [
        _msg_kernels[t](k2[t][1], k2[t][2], sc_r, dst_r) for t in range(S)
    ]

    k3 = [
        _tc_layer3(a2[t][0], a2[t][1], k2[t][0],
                   deg_col[t], b_row[t][1], al_row[t], batch_col)
        for t in range(S)
    ]

    z_T = jnp.stack([k3[t][0][:N] for t in range(S)])
    g_T = jnp.stack([jnp.concatenate([k2[t][3], k3[t][1]], axis=1)
                     for t in range(S)])
    return (z_T, g_T)
